# Initial kernel scaffold; baseline (speedup 1.0000x reference)
import jax
import jax.numpy as jnp
from jax.experimental import pallas as pl


def kernel(input, span_idxs, W, b):
    raise NotImplementedError("write your pallas kernel here")


# speedup vs baseline: 99.2385x; 99.2385x over previous
import functools

import jax
import jax.numpy as jnp
from jax import lax
from jax.experimental import pallas as pl
from jax.experimental.pallas import tpu as pltpu
from jax.experimental.pallas import tpu_sc as plsc

_B, _S, _L, _V, _D = 4096, 16, 200, 1000, 128
_VP = 1008
_NC, _NS = 2, 16
_NW = _NC * _NS
_BPW = _B // _NW
_NR = _B * _S
_RB = 256


def _sc_body(tok_hbm, span_hbm, bow_hbm, tokv, spanv,
             buf0, buf1, buf2, buf3, sem0, sem1, sem2, sem3):
  bufs = (buf0, buf1, buf2, buf3)
  sems = (sem0, sem1, sem2, sem3)
  wid = lax.axis_index("s") * _NC + lax.axis_index("c")
  b0 = wid * _BPW

  pltpu.sync_copy(tok_hbm.at[pl.ds(b0, _BPW)], tokv)
  pltpu.sync_copy(span_hbm.at[pl.ds(b0, _BPW)], spanv)

  neg1 = jnp.full((16,), -1, jnp.int32)
  for buf in bufs:
    for c in range(_VP // 16):
      buf[pl.ds(c * 16, 16)] = neg1

  iota16 = lax.broadcasted_iota(jnp.int32, (16,), 0)

  def bloop(i, _):
    sp_a = spanv[i, pl.ds(0, 16)]
    sp_b = spanv[i, pl.ds(16, 16)]
    for s in range(_S):
      k = s % 4
      buf = bufs[k]
      sem = sems[k]
      row = (b0 + i) * _S + s

      if s >= 4:
        pltpu.make_async_copy(buf, bow_hbm.at[0], sem).wait()
      else:
        @pl.when(i > 0)
        def _():
          pltpu.make_async_copy(buf, bow_hbm.at[0], sem).wait()

      sp = sp_a if s < 8 else sp_b
      lo = sp[(2 * s) % 16]
      hi = sp[(2 * s) % 16 + 1]
      sid = jnp.full((16,), row, jnp.int32)
      for c in range(13):
        off = 184 if c == 12 else c * 16
        tokc = tokv[i, pl.ds(off, 16)]
        pos = iota16 + off
        msk = (pos >= lo) & (pos < hi)
        plsc.store_scatter(buf, [tokc], sid, mask=msk)

      pltpu.async_copy(buf, bow_hbm.at[row], sem)
    return _

  lax.fori_loop(0, _BPW, bloop, None)
  for k in range(4):
    pltpu.make_async_copy(bufs[k], bow_hbm.at[0], sems[k]).wait()


@jax.jit
def _sc_scatter(tok, span2):
  mesh = plsc.VectorSubcoreMesh(core_axis_name="c", subcore_axis_name="s")
  return pl.kernel(
      _sc_body,
      out_type=jax.ShapeDtypeStruct((_NR, _VP), jnp.int32),
      mesh=mesh,
      compiler_params=pltpu.CompilerParams(needs_layout_passes=False),
      scratch_types=[
          pltpu.VMEM((_BPW, _L), jnp.int32),
          pltpu.VMEM((_BPW, 2 * _S), jnp.int32),
          pltpu.VMEM((_VP,), jnp.int32),
          pltpu.VMEM((_VP,), jnp.int32),
          pltpu.VMEM((_VP,), jnp.int32),
          pltpu.VMEM((_VP,), jnp.int32),
          pltpu.SemaphoreType.DMA,
          pltpu.SemaphoreType.DMA,
          pltpu.SemaphoreType.DMA,
          pltpu.SemaphoreType.DMA,
      ],
  )(tok, span2)


def _mm_body(bow_ref, wt_ref, b_ref, o_ref):
  rid = pl.program_id(0) * _RB + lax.broadcasted_iota(jnp.int32, (_RB, 1), 0)
  bow = (bow_ref[...] == rid).astype(jnp.float32)
  o_ref[...] = (
      jnp.dot(bow, wt_ref[...], preferred_element_type=jnp.float32)
      + b_ref[...]
  )


@jax.jit
def _mm(bow, wt, bias):
  return pl.pallas_call(
      _mm_body,
      grid=(_NR // _RB,),
      in_specs=[
          pl.BlockSpec((_RB, _VP), lambda i: (i, 0)),
          pl.BlockSpec((_VP, _D), lambda i: (0, 0)),
          pl.BlockSpec((1, _D), lambda i: (0, 0)),
      ],
      out_specs=pl.BlockSpec((_RB, _D), lambda i: (i, 0)),
      out_shape=jax.ShapeDtypeStruct((_NR, _D), jnp.float32),
  )(bow, wt, bias)


def kernel(input, span_idxs, W, b):
  span2 = span_idxs.reshape(_B, 2 * _S)
  bow = _sc_scatter(input, span2)
  wt = jnp.concatenate(
      [W.T, jnp.zeros((_VP - _V, _D), jnp.float32)], axis=0)
  out = _mm(bow, wt, b.reshape(1, _D))
  return out.reshape(_B, _S, _D)
